# baseline (speedup 1.0000x reference)
import functools

import jax
import jax.numpy as jnp
from jax import lax
from jax.experimental import pallas as pl
from jax.experimental.pallas import tpu as pltpu
from jax.experimental.pallas import tpu_sc as plsc

_N = 2048
_C = 128
_CAP = 272
_TRIG = 240
_NW = 32
_AG = _N // _NW
_ATC = 256
_BIGF = 3e38
_PADXY = 1e9


def _sc_select(sT):
    mesh = plsc.VectorSubcoreMesh(core_axis_name="c", subcore_axis_name="s")

    @functools.partial(
        pl.kernel,
        mesh=mesh,
        out_type=jax.ShapeDtypeStruct((5, _N, _C), jnp.float32),
        scratch_types=[
            pltpu.VMEM((_N,), jnp.float32),
            pltpu.VMEM((_N,), jnp.float32),
            pltpu.VMEM((_N,), jnp.float32),
            pltpu.VMEM((_N,), jnp.float32),
            pltpu.VMEM((_N,), jnp.float32),
            pltpu.VMEM((_N,), jnp.int32),
            pltpu.VMEM((_C,), jnp.int32),
            pltpu.VMEM((5, _AG, _C), jnp.float32),
            pltpu.SemaphoreType.DMA,
        ],
        compiler_params=pltpu.CompilerParams(needs_layout_passes=False),
    )
    def sel(sT_hbm, out_hbm, sx, sy, svx, svy, ckey0, cidx0, cidxf, xout, sem):
        wid = lax.axis_index("s") * 2 + lax.axis_index("c")
        base = wid * _AG
        pltpu.sync_copy(sT_hbm.at[0], sx)
        pltpu.sync_copy(sT_hbm.at[1], sy)
        pltpu.sync_copy(sT_hbm.at[2], svx)
        pltpu.sync_copy(sT_hbm.at[3], svy)
        lanes = lax.broadcasted_iota(jnp.int32, (16,), 0)

        def count_le(thr, nch, cnt0):
            def cbody(cc, acc):
                kv = ckey0[pl.ds(cc * 16, 16)]
                mm = (kv <= thr) & ((cc * 16 + lanes) < cnt0)
                return acc + mm.astype(jnp.int32)
            acc = lax.fori_loop(0, nch, cbody, jnp.zeros((16,), jnp.int32))
            return jnp.sum(acc)

        def per_agent(a, _):
            i = base + a
            isplat = i + jnp.zeros((16,), jnp.int32)
            six = plsc.load_gather(sx, [isplat])
            siy = plsc.load_gather(sy, [isplat])
            sivx = plsc.load_gather(svx, [isplat])
            sivy = plsc.load_gather(svy, [isplat])
            bound0 = jnp.float32(1.0 + 3e-6)

            def kchunk(c, bsplat):
                kx = sx[pl.ds(c * 16, 16)]
                ky = sy[pl.ds(c * 16, 16)]
                dx = six - kx
                dy = siy - ky
                keyv = (dx * dx + 1e-6) + (dy * dy + 1e-6)
                m = keyv <= bound0
                pos = bsplat + plsc.cumsum(m.astype(jnp.int32)) - 1
                plsc.store_scatter(ckey0, [pos], keyv, mask=m)
                plsc.store_scatter(cidx0, [pos], c * 16 + lanes, mask=m)
                return bsplat + plsc.all_reduce_population_count(m)

            bs0 = plsc.parallel_loop(0, _N // 16, unroll=8,
                                     carry=jnp.zeros((16,), jnp.int32))(
                                         kchunk)
            c0 = jnp.max(bs0)
            nch = (c0 + 15) >> 4

            def r_cond(st):
                _, _, c, it = st
                return (c > _C) & (it < 24)

            def r_body(st):
                vlo, vhi, c, it = st
                mid = vlo + ((vhi - vlo) >> 1)
                thr16 = plsc.bitcast(mid + jnp.zeros((16,), jnp.int32),
                                     jnp.float32)
                thr = jnp.max(thr16)
                cm = count_le(thr, nch, c0)
                ok = cm >= 64
                vhi = jnp.where(ok, mid, vhi)
                vlo = jnp.where(ok, vlo, mid + 1)
                c = jnp.where(ok, cm, c)
                return (vlo, vhi, c, it + 1)

            b0bits = jnp.int32(0x3F800019)
            _, vhib, cnt, _ = lax.while_loop(
                r_cond, r_body, (jnp.int32(0), b0bits, c0, jnp.int32(0)))
            bound16 = plsc.bitcast(vhib + jnp.zeros((16,), jnp.int32),
                                   jnp.float32)
            bound = jnp.max(bound16)

            def cchunk(c, bsplat):
                kv = ckey0[pl.ds(c * 16, 16)]
                iv0 = cidx0[pl.ds(c * 16, 16)]
                m = (kv <= bound) & ((c * 16 + lanes) < c0)
                pos = bsplat + plsc.cumsum(m.astype(jnp.int32)) - 1
                pos = jnp.minimum(pos, _C - 1)
                plsc.store_scatter(cidxf, [pos], iv0, mask=m)
                return bsplat + plsc.all_reduce_population_count(m)

            bsplat = lax.fori_loop(0, nch, cchunk,
                                   jnp.zeros((16,), jnp.int32))
            cnt = jnp.minimum(jnp.max(bsplat), _C)

            for cc in range(_C // 16):
                off = cc * 16
                valid = (off + lanes) < cnt
                iv = cidxf[pl.ds(off, 16)]
                iv = jnp.where(valid, iv, i)
                gx = plsc.load_gather(sx, [iv])
                gy = plsc.load_gather(sy, [iv])
                gvx = plsc.load_gather(svx, [iv])
                gvy = plsc.load_gather(svy, [iv])
                xout[0, a, pl.ds(off, 16)] = jnp.where(
                    valid, six - gx, jnp.float32(_PADXY))
                xout[1, a, pl.ds(off, 16)] = jnp.where(
                    valid, siy - gy, jnp.float32(_PADXY))
                xout[2, a, pl.ds(off, 16)] = jnp.where(valid, sivx - gvx, 0.0)
                xout[3, a, pl.ds(off, 16)] = jnp.where(valid, sivy - gvy, 0.0)
                xout[4, a, pl.ds(off, 16)] = jnp.where(
                    valid & (iv == i), 1.0, 0.0)
            return 0

        lax.fori_loop(0, _AG, per_agent, 0)
        for ch in range(5):
            pltpu.sync_copy(xout.at[ch], out_hbm.at[ch, pl.ds(base, _AG)])

    return sel(sT)


def _tc_body(x_ref, s_ref, g_ref, W1, b1, W2, b2, Wf1, bf1, Wf2, bf2,
             Wf3, bf3, Wf4, bf4, out_ref):
    A = _ATC
    X = x_ref[...].reshape(5, A * _C)
    dxf = x_ref[0]
    dyf = x_ref[1]
    key = (dxf * dxf + 1e-6) + (dyf * dyf + 1e-6)
    kb = lax.bitcast_convert_type(key, jnp.int32)
    kbT = kb.T
    onesC = jnp.ones((1, _C), jnp.float32)

    def bs(_, lohi):
        lo, hi = lohi
        mid = lo + ((hi - lo) >> 1)
        cmp = (kbT <= mid).astype(jnp.float32)
        cnt = jnp.dot(onesC, cmp, preferred_element_type=jnp.float32)
        ok = cnt >= 64.0
        return (jnp.where(ok, lo, mid + 1), jnp.where(ok, mid, hi))

    lo0 = jnp.full((1, A), jnp.int32(0x35863268))
    hi0 = jnp.full((1, A), jnp.int32(0x3F800019))
    _, thr = lax.fori_loop(0, 28, bs, (lo0, hi0))
    mtop = kb <= thr.T
    mrad = jnp.sqrt(dxf * dxf + dyf * dyf) < 1.0
    msk = (mtop & mrad).astype(jnp.float32)

    h = jnp.maximum(
        lax.dot_general(X, W1[...], (((0,), (0,)), ((), ())),
                        preferred_element_type=jnp.float32) + b1[...], 0.0)
    h = jnp.maximum(jnp.dot(h, W2[...],
                            preferred_element_type=jnp.float32) + b2[...], 0.0)
    h3 = h.reshape(A, _C, 128) * msk[:, :, None]
    hm = jnp.max(h3, axis=1)

    s = s_ref[...]
    sg = s[:, :2] - g_ref[...]
    feat = jnp.concatenate([hm, sg, s[:, 2:]], axis=1)
    y = jnp.maximum(jnp.dot(feat, Wf1[...],
                            preferred_element_type=jnp.float32) + bf1[...], 0.0)
    y = jnp.maximum(jnp.dot(y, Wf2[...],
                            preferred_element_type=jnp.float32) + bf2[...], 0.0)
    y = jnp.maximum(jnp.dot(y, Wf3[...],
                            preferred_element_type=jnp.float32) + bf3[...], 0.0)
    y = jnp.dot(y, Wf4[...],
                preferred_element_type=jnp.float32) + bf4[...]
    y = 2.0 * jax.nn.sigmoid(y) + 0.2
    a_x = -(y[:, 0:1] * sg[:, 0:1] + y[:, 1:2] * s[:, 2:3])
    a_y = -(y[:, 2:3] * sg[:, 1:2] + y[:, 3:4] * s[:, 3:4])
    out_ref[...] = jnp.concatenate([a_x, a_y], axis=1)


def _tc_stage(xcand, s, g, W1, b1, W2, b2, Wf1, bf1, Wf2, bf2, Wf3,
              bf3, Wf4, bf4):
    grid = (_N // _ATC,)
    full = lambda arr: pl.BlockSpec(arr.shape, lambda i: (0,) * arr.ndim)
    in_specs = [
        pl.BlockSpec((5, _ATC, _C), lambda i: (0, i, 0)),
        pl.BlockSpec((_ATC, 4), lambda i: (i, 0)),
        pl.BlockSpec((_ATC, 2), lambda i: (i, 0)),
    ] + [full(w) for w in (W1, b1, W2, b2, Wf1, bf1, Wf2, bf2, Wf3,
                           bf3, Wf4, bf4)]
    return pl.pallas_call(
        _tc_body,
        grid=grid,
        in_specs=in_specs,
        out_specs=pl.BlockSpec((_ATC, 2), lambda i: (i, 0)),
        out_shape=jax.ShapeDtypeStruct((_N, 2), jnp.float32),
    )(xcand, s, g, W1, b1, W2, b2, Wf1, bf1, Wf2, bf2, Wf3, bf3,
      Wf4, bf4)


def kernel(s, g, W1, b1, W2, b2, Wf1, bf1, Wf2, bf2, Wf3, bf3, Wf4, bf4):
    sT = s.T
    xcand = _sc_select(sT)
    return _tc_stage(
        xcand, s, g,
        W1, b1.reshape(1, -1), W2, b2.reshape(1, -1),
        Wf1, bf1.reshape(1, -1), Wf2, bf2.reshape(1, -1),
        Wf3, bf3.reshape(1, -1), Wf4, bf4.reshape(1, -1))


# speedup vs baseline: 1.2128x; 1.2128x over previous
import functools

import jax
import jax.numpy as jnp
from jax import lax
from jax.experimental import pallas as pl
from jax.experimental.pallas import tpu as pltpu
from jax.experimental.pallas import tpu_sc as plsc

_N = 2048
_C = 128
_CAP = 272
_TRIG = 240
_NW = 32
_AG = _N // _NW
_ATC = 256
_BIGF = 3e38
_PADXY = 1e9


def _sc_select(sT):
    mesh = plsc.VectorSubcoreMesh(core_axis_name="c", subcore_axis_name="s")

    @functools.partial(
        pl.kernel,
        mesh=mesh,
        out_type=jax.ShapeDtypeStruct((5, _N, _C), jnp.float32),
        scratch_types=[
            pltpu.VMEM((_N,), jnp.float32),
            pltpu.VMEM((_N,), jnp.float32),
            pltpu.VMEM((_N,), jnp.float32),
            pltpu.VMEM((_N,), jnp.float32),
            pltpu.VMEM((_N,), jnp.float32),
            pltpu.VMEM((_N,), jnp.int32),
            pltpu.VMEM((_C,), jnp.int32),
            pltpu.VMEM((5, _AG, _C), jnp.float32),
            pltpu.SemaphoreType.DMA,
        ],
        compiler_params=pltpu.CompilerParams(needs_layout_passes=False),
    )
    def sel(sT_hbm, out_hbm, sx, sy, svx, svy, ckey0, cidx0, cidxf, xout, sem):
        wid = lax.axis_index("s") * 2 + lax.axis_index("c")
        base = wid * _AG
        pltpu.sync_copy(sT_hbm.at[0], sx)
        pltpu.sync_copy(sT_hbm.at[1], sy)
        pltpu.sync_copy(sT_hbm.at[2], svx)
        pltpu.sync_copy(sT_hbm.at[3], svy)
        lanes = lax.broadcasted_iota(jnp.int32, (16,), 0)

        def count_le(thr, nch, cnt0):
            def cbody(cc, acc):
                kv = ckey0[pl.ds(cc * 16, 16)]
                mm = (kv <= thr) & ((cc * 16 + lanes) < cnt0)
                return acc + mm.astype(jnp.int32)
            acc = plsc.parallel_loop(0, nch, unroll=4,
                                     carry=jnp.zeros((16,), jnp.int32))(cbody)
            return jnp.sum(acc)

        def per_agent(a, _):
            i = base + a
            isplat = i + jnp.zeros((16,), jnp.int32)
            six = plsc.load_gather(sx, [isplat])
            siy = plsc.load_gather(sy, [isplat])
            sivx = plsc.load_gather(svx, [isplat])
            sivy = plsc.load_gather(svy, [isplat])
            bound0 = jnp.float32(1.0 + 3e-6)

            def kchunk(c, bsplat):
                kx = sx[pl.ds(c * 16, 16)]
                ky = sy[pl.ds(c * 16, 16)]
                dx = six - kx
                dy = siy - ky
                keyv = (dx * dx + 1e-6) + (dy * dy + 1e-6)
                m = keyv <= bound0
                pos = bsplat + plsc.cumsum(m.astype(jnp.int32)) - 1
                plsc.store_scatter(ckey0, [pos], keyv, mask=m)
                plsc.store_scatter(cidx0, [pos], c * 16 + lanes, mask=m)
                return bsplat + plsc.all_reduce_population_count(m)

            bs0 = plsc.parallel_loop(0, _N // 16, unroll=8,
                                     carry=jnp.zeros((16,), jnp.int32))(
                                         kchunk)
            c0 = jnp.max(bs0)
            nch = (c0 + 15) >> 4

            def r_cond(st):
                _, _, c, it = st
                return (c > _C) & (it < 24)

            def r_body(st):
                vlo, vhi, c, it = st
                mid = vlo + ((vhi - vlo) >> 1)
                thr16 = plsc.bitcast(mid + jnp.zeros((16,), jnp.int32),
                                     jnp.float32)
                thr = jnp.max(thr16)
                cm = count_le(thr, nch, c0)
                ok = cm >= 64
                vhi = jnp.where(ok, mid, vhi)
                vlo = jnp.where(ok, vlo, mid + 1)
                c = jnp.where(ok, cm, c)
                return (vlo, vhi, c, it + 1)

            b0bits = jnp.int32(0x3F800019)
            _, vhib, cnt, _ = lax.while_loop(
                r_cond, r_body, (jnp.int32(0), b0bits, c0, jnp.int32(0)))
            bound16 = plsc.bitcast(vhib + jnp.zeros((16,), jnp.int32),
                                   jnp.float32)
            bound = jnp.max(bound16)

            def cchunk(c, bsplat):
                kv = ckey0[pl.ds(c * 16, 16)]
                iv0 = cidx0[pl.ds(c * 16, 16)]
                m = (kv <= bound) & ((c * 16 + lanes) < c0)
                pos = bsplat + plsc.cumsum(m.astype(jnp.int32)) - 1
                pos = jnp.minimum(pos, _C - 1)
                plsc.store_scatter(cidxf, [pos], iv0, mask=m)
                return bsplat + plsc.all_reduce_population_count(m)

            bsplat = plsc.parallel_loop(0, nch, unroll=4,
                                         carry=jnp.zeros((16,), jnp.int32))(
                                             cchunk)
            cnt = jnp.minimum(jnp.max(bsplat), _C)

            for cc in range(_C // 16):
                off = cc * 16
                valid = (off + lanes) < cnt
                iv = cidxf[pl.ds(off, 16)]
                iv = jnp.where(valid, iv, i)
                gx = plsc.load_gather(sx, [iv])
                gy = plsc.load_gather(sy, [iv])
                gvx = plsc.load_gather(svx, [iv])
                gvy = plsc.load_gather(svy, [iv])
                xout[0, a, pl.ds(off, 16)] = jnp.where(
                    valid, six - gx, jnp.float32(_PADXY))
                xout[1, a, pl.ds(off, 16)] = jnp.where(
                    valid, siy - gy, jnp.float32(_PADXY))
                xout[2, a, pl.ds(off, 16)] = jnp.where(valid, sivx - gvx, 0.0)
                xout[3, a, pl.ds(off, 16)] = jnp.where(valid, sivy - gvy, 0.0)
                xout[4, a, pl.ds(off, 16)] = jnp.where(
                    valid & (iv == i), 1.0, 0.0)
            return 0

        lax.fori_loop(0, _AG, per_agent, 0)
        for ch in range(5):
            pltpu.sync_copy(xout.at[ch], out_hbm.at[ch, pl.ds(base, _AG)])

    return sel(sT)


def _tc_body(x_ref, s_ref, g_ref, W1, b1, W2, b2, Wf1, bf1, Wf2, bf2,
             Wf3, bf3, Wf4, bf4, out_ref):
    A = _ATC
    X = x_ref[...].reshape(5, A * _C)
    dxf = x_ref[0]
    dyf = x_ref[1]
    key = (dxf * dxf + 1e-6) + (dyf * dyf + 1e-6)
    kb = lax.bitcast_convert_type(key, jnp.int32)
    kbT = kb.T
    onesC = jnp.ones((1, _C), jnp.float32)

    def bs(_, lohi):
        lo, hi = lohi
        mid = lo + ((hi - lo) >> 1)
        cmp = (kbT <= mid).astype(jnp.float32)
        cnt = jnp.dot(onesC, cmp, preferred_element_type=jnp.float32)
        ok = cnt >= 64.0
        return (jnp.where(ok, lo, mid + 1), jnp.where(ok, mid, hi))

    lo0 = jnp.full((1, A), jnp.int32(0x35863268))
    hi0 = jnp.full((1, A), jnp.int32(0x3F800019))
    _, thr = lax.fori_loop(0, 28, bs, (lo0, hi0))
    mtop = kb <= thr.T
    mrad = jnp.sqrt(dxf * dxf + dyf * dyf) < 1.0
    msk = (mtop & mrad).astype(jnp.float32)

    h = jnp.maximum(
        lax.dot_general(X, W1[...], (((0,), (0,)), ((), ())),
                        preferred_element_type=jnp.float32) + b1[...], 0.0)
    h = jnp.maximum(jnp.dot(h, W2[...],
                            preferred_element_type=jnp.float32) + b2[...], 0.0)
    h3 = h.reshape(A, _C, 128) * msk[:, :, None]
    hm = jnp.max(h3, axis=1)

    s = s_ref[...]
    sg = s[:, :2] - g_ref[...]
    feat = jnp.concatenate([hm, sg, s[:, 2:]], axis=1)
    y = jnp.maximum(jnp.dot(feat, Wf1[...],
                            preferred_element_type=jnp.float32) + bf1[...], 0.0)
    y = jnp.maximum(jnp.dot(y, Wf2[...],
                            preferred_element_type=jnp.float32) + bf2[...], 0.0)
    y = jnp.maximum(jnp.dot(y, Wf3[...],
                            preferred_element_type=jnp.float32) + bf3[...], 0.0)
    y = jnp.dot(y, Wf4[...],
                preferred_element_type=jnp.float32) + bf4[...]
    y = 2.0 * jax.nn.sigmoid(y) + 0.2
    a_x = -(y[:, 0:1] * sg[:, 0:1] + y[:, 1:2] * s[:, 2:3])
    a_y = -(y[:, 2:3] * sg[:, 1:2] + y[:, 3:4] * s[:, 3:4])
    out_ref[...] = jnp.concatenate([a_x, a_y], axis=1)


def _tc_stage(xcand, s, g, W1, b1, W2, b2, Wf1, bf1, Wf2, bf2, Wf3,
              bf3, Wf4, bf4):
    grid = (_N // _ATC,)
    full = lambda arr: pl.BlockSpec(arr.shape, lambda i: (0,) * arr.ndim)
    in_specs = [
        pl.BlockSpec((5, _ATC, _C), lambda i: (0, i, 0)),
        pl.BlockSpec((_ATC, 4), lambda i: (i, 0)),
        pl.BlockSpec((_ATC, 2), lambda i: (i, 0)),
    ] + [full(w) for w in (W1, b1, W2, b2, Wf1, bf1, Wf2, bf2, Wf3,
                           bf3, Wf4, bf4)]
    return pl.pallas_call(
        _tc_body,
        grid=grid,
        in_specs=in_specs,
        out_specs=pl.BlockSpec((_ATC, 2), lambda i: (i, 0)),
        out_shape=jax.ShapeDtypeStruct((_N, 2), jnp.float32),
    )(xcand, s, g, W1, b1, W2, b2, Wf1, bf1, Wf2, bf2, Wf3, bf3,
      Wf4, bf4)


def kernel(s, g, W1, b1, W2, b2, Wf1, bf1, Wf2, bf2, Wf3, bf3, Wf4, bf4):
    sT = s.T
    xcand = _sc_select(sT)
    return _tc_stage(
        xcand, s, g,
        W1, b1.reshape(1, -1), W2, b2.reshape(1, -1),
        Wf1, bf1.reshape(1, -1), Wf2, bf2.reshape(1, -1),
        Wf3, bf3.reshape(1, -1), Wf4, bf4.reshape(1, -1))
